# baseline (speedup 1.0000x reference)
import functools

import jax
import jax.numpy as jnp
from jax import lax
from jax.experimental import pallas as pl
from jax.experimental.pallas import tpu as pltpu
from jax.experimental.pallas import tpu_sc as plsc

_DIM = 128
_B = 512
_MARGIN = 0.1
_MARGIN1 = 0.05
_INF = 4.0
_NW = 32
_BPW = _B // _NW
_NCE = 14
_NRE = 2
_ROWS = 21 * _BPW
_PROWS = 48
_L = 16


@functools.cache
def _get_sc_kernel():
    mesh = plsc.VectorSubcoreMesh(core_axis_name="c", subcore_axis_name="s")

    @functools.partial(
        pl.kernel,
        mesh=mesh,
        out_type=jax.ShapeDtypeStruct((_NW * _PROWS, 128), jnp.float32),
        scratch_types=[
            pltpu.VMEM((32,), jnp.int32),
            pltpu.VMEM((48,), jnp.int32),
            pltpu.VMEM((48,), jnp.int32),
            pltpu.VMEM((48,), jnp.int32),
            pltpu.VMEM((32,), jnp.int32),
            pltpu.VMEM((32,), jnp.int32),
            pltpu.VMEM((_BPW,), jnp.int32),
            pltpu.VMEM((_BPW,), jnp.int32),
            pltpu.VMEM((_BPW,), jnp.int32),
            pltpu.VMEM((32, 2 * _DIM), jnp.float32),
            pltpu.VMEM((48, 2 * _DIM), jnp.float32),
            pltpu.VMEM((48, 2 * _DIM), jnp.float32),
            pltpu.VMEM((48, 2 * _DIM), jnp.float32),
            pltpu.VMEM((32, 2 * _DIM), jnp.float32),
            pltpu.VMEM((32, 2 * _DIM), jnp.float32),
            pltpu.VMEM((_BPW, 2 * _DIM), jnp.float32),
            pltpu.VMEM((_BPW, 2 * _DIM), jnp.float32),
            pltpu.VMEM((_BPW, 2 * _DIM), jnp.float32),
            pltpu.VMEM((_PROWS, 128), jnp.float32),
            pltpu.SemaphoreType.DMA,
        ],
    )
    def _sc_loss(ce_hbm, re_hbm, nf1_hbm, nf2_hbm, nf3_hbm, nf4_hbm,
                 dis_hbm, neg_hbm, top_hbm, rel_hbm, out_hbm,
                 i1, i2, i3, i4, i5, i6, i7, ir3, ir4,
                 r1v, r2v, r3v, r4v, r5v, r6v, r7v, rr3, rr4, buf, sem):
        w = lax.axis_index("s") * 2 + lax.axis_index("c")

        pltpu.sync_copy(nf1_hbm.at[pl.ds(32 * w, 32)], i1)
        g1 = pltpu.async_copy(ce_hbm.at[i1], r1v, sem)
        pltpu.sync_copy(nf2_hbm.at[pl.ds(48 * w, 48)], i2)
        g2 = pltpu.async_copy(ce_hbm.at[i2], r2v, sem)
        pltpu.sync_copy(nf3_hbm.at[pl.ds(48 * w, 48)], i3)
        g3 = pltpu.async_copy(ce_hbm.at[i3], r3v, sem)
        pltpu.sync_copy(rel_hbm.at[0, pl.ds(_BPW * w, _BPW)], ir3)
        gr3 = pltpu.async_copy(re_hbm.at[ir3], rr3, sem)
        pltpu.sync_copy(nf4_hbm.at[pl.ds(48 * w, 48)], i4)
        g4 = pltpu.async_copy(ce_hbm.at[i4], r4v, sem)
        pltpu.sync_copy(rel_hbm.at[1, pl.ds(_BPW * w, _BPW)], ir4)
        gr4 = pltpu.async_copy(re_hbm.at[ir4], rr4, sem)
        pltpu.sync_copy(dis_hbm.at[pl.ds(32 * w, 32)], i5)
        g5 = pltpu.async_copy(ce_hbm.at[i5], r5v, sem)
        pltpu.sync_copy(neg_hbm.at[pl.ds(32 * w, 32)], i6)
        g6 = pltpu.async_copy(ce_hbm.at[i6], r6v, sem)
        pltpu.sync_copy(top_hbm.at[pl.ds(_BPW * w, _BPW)], i7)
        g7 = pltpu.async_copy(ce_hbm.at[i7], r7v, sem)

        m = jnp.float32(_MARGIN)
        m1 = jnp.float32(_MARGIN1)
        lanes = lax.iota(jnp.int32, _L)

        def halves(ref, row, g):
            a = ref[row, pl.ds(g * _L, _L)]
            b = jnp.abs(ref[row, pl.ds(_DIM + g * _L, _L)])
            return a, b

        def put(row, acc_or_parts):
            if isinstance(acc_or_parts, list):
                acc = acc_or_parts[0] * acc_or_parts[0]
                for q in acc_or_parts[1:]:
                    acc = acc + q * q
            else:
                acc = acc_or_parts
            buf[row // 8, pl.ds((row % 8) * _L, _L)] = acc

        def head_2op(base, ref, sgn_r, bias):
            def body(i, _):
                t1 = []
                t2 = []
                t3 = []
                for g in range(8):
                    c1, cr = halves(ref, 2 * i, g)
                    d1, dr = halves(ref, 2 * i + 1, g)
                    euc = jnp.abs(c1 - d1)
                    if sgn_r == 0:
                        t = euc + cr - dr + bias
                    elif sgn_r == 1:
                        t = cr + dr + bias - euc
                    else:
                        t = euc - cr - dr + bias
                    t1.append(jnp.maximum(t, 0.0))
                    t2.append(jnp.maximum(m - cr, 0.0))
                    t3.append(jnp.maximum(m - dr, 0.0))
                put(base + i, t1)
                put(base + _BPW + i, t2)
                put(base + 2 * _BPW + i, t3)
                return 0

            lax.fori_loop(0, _BPW, body, 0, unroll=False)

        g1.wait()
        head_2op(0, r1v, 0, m1)

        def nf2_body(i, _):
            t1 = []
            t2 = []
            for g in range(8):
                c1, c2 = halves(r2v, 3 * i, g)
                d1, d2 = halves(r2v, 3 * i + 1, g)
                e1, er = halves(r2v, 3 * i + 2, g)
                start = jnp.maximum(c1 - c2, d1 - d2)
                end = jnp.minimum(c1 + c2, d1 + d2)
                new_r = (end - start) * 0.5
                cen = (start + end) * 0.5
                euc = jnp.abs(cen - e1)
                t1.append(jnp.maximum(euc + new_r - er + m1, 0.0))
                t2.append(jnp.maximum(start - end, 0.0))
            put(9 * _BPW + i, t1)
            put(10 * _BPW + i, t2)
            return 0

        g2.wait()
        lax.fori_loop(0, _BPW, nf2_body, 0, unroll=False)

        def rel_head(base, ref, rref, crow, drow, sgn):
            def body(i, _):
                dtail = rref[i, pl.ds(_DIM - _L + 1, _L)]
                delta = jnp.abs(dtail[_L - 1])
                bias = m1 - delta if sgn > 0 else m1 + delta
                t1 = []
                t2 = []
                t3 = []
                for g in range(8):
                    c1, cr = halves(ref, 3 * i + crow, g)
                    d1, dr = halves(ref, 3 * i + drow, g)
                    r1 = rref[i, pl.ds(g * _L, _L)]
                    euc = jnp.abs(c1 + r1 - d1) if sgn > 0 else jnp.abs(c1 - r1 - d1)
                    if sgn > 0:
                        t = euc + cr - dr + bias
                    else:
                        t = euc - cr - dr + bias
                    t1.append(jnp.maximum(t, 0.0))
                    t2.append(jnp.maximum(m - cr, 0.0))
                    t3.append(jnp.maximum(m - dr, 0.0))
                put(base + i, t1)
                put(base + _BPW + i, t2)
                put(base + 2 * _BPW + i, t3)
                put(base + 3 * _BPW + i,
                    jnp.where(lanes == _L - 1, dtail * dtail, 0.0))
                return 0

            lax.fori_loop(0, _BPW, body, 0, unroll=False)

        g3.wait()
        gr3.wait()
        rel_head(11 * _BPW, r3v, rr3, 0, 2, 1)
        g4.wait()
        gr4.wait()
        rel_head(15 * _BPW, r4v, rr4, 1, 2, -1)

        g5.wait()
        head_2op(3 * _BPW, r5v, 1, m1)
        g6.wait()
        head_2op(6 * _BPW, r6v, -1, -m1)

        def top_body(i, _):
            t1 = []
            t2 = []
            for g in range(8):
                d1, dr = halves(r7v, i, g)
                t1.append(jnp.maximum(_INF - dr * 0.5, 0.0))
                t2.append(jnp.maximum(_INF + d1, 0.0))
            put(19 * _BPW + i, t1)
            put(20 * _BPW + i, t2)
            return 0

        g7.wait()
        lax.fori_loop(0, _BPW, top_body, 0, unroll=False)

        zero = jnp.zeros((_L,), jnp.float32)
        for r in range(_ROWS, _PROWS * 8):
            buf[r // 8, pl.ds((r % 8) * _L, _L)] = zero
        pltpu.sync_copy(buf, out_hbm.at[pl.ds(w * _PROWS, _PROWS)])

    return _sc_loss


def _finalize_body(p_ref, out_ref):
    x = p_ref[...]
    col = lax.broadcasted_iota(jnp.int32, (128, 8), 0) // _L
    grp = lax.broadcasted_iota(jnp.int32, (128, 8), 1)
    sel = (col == grp).astype(jnp.float32)
    x_hi = x.astype(jnp.bfloat16).astype(jnp.float32)
    x_lo = x - x_hi
    dn = (((1,), (0,)), ((), ()))
    sums = (lax.dot_general(x_hi, sel, dn, preferred_element_type=jnp.float32)
            + lax.dot_general(x_lo, sel, dn, preferred_element_type=jnp.float32))
    out_ref[0, 0] = jnp.sum(jnp.sqrt(sums)) * (1.0 / _B)


def _finalize(partials):
    return pl.pallas_call(
        _finalize_body,
        out_shape=jax.ShapeDtypeStruct((1, 1), jnp.float32),
        out_specs=pl.BlockSpec(memory_space=pltpu.SMEM),
    )(partials)


def kernel(class_emb, rel_emb, nf1, nf2, nf3, nf4, disjoint, neg, top):
    re_pad = jnp.pad(rel_emb, ((0, 0), (0, 2 * _DIM - (_DIM + 1))))
    rel_cols = jnp.stack([nf3[:_B, 1], nf4[:_B, 0]])
    partials = _get_sc_kernel()(
        class_emb, re_pad,
        nf1.reshape(-1), nf2.reshape(-1), nf3.reshape(-1), nf4.reshape(-1),
        disjoint.reshape(-1), neg.reshape(-1), top, rel_cols)
    return _finalize(partials)[0, 0]


# speedup vs baseline: 1.0447x; 1.0447x over previous
import functools

import jax
import jax.numpy as jnp
from jax import lax
from jax.experimental import pallas as pl
from jax.experimental.pallas import tpu as pltpu
from jax.experimental.pallas import tpu_sc as plsc

_DIM = 128
_B = 512
_MARGIN = 0.1
_MARGIN1 = 0.05
_INF = 4.0
_NW = 32
_BPW = _B // _NW
_NCE = 14
_NRE = 2
_ROWS = 21 * _BPW
_PROWS = 48
_L = 16


@functools.cache
def _get_sc_kernel():
    mesh = plsc.VectorSubcoreMesh(core_axis_name="c", subcore_axis_name="s")

    @functools.partial(
        pl.kernel,
        mesh=mesh,
        out_type=jax.ShapeDtypeStruct((_NW * _PROWS, 128), jnp.float32),
        scratch_types=[
            pltpu.VMEM((32,), jnp.int32),
            pltpu.VMEM((48,), jnp.int32),
            pltpu.VMEM((48,), jnp.int32),
            pltpu.VMEM((48,), jnp.int32),
            pltpu.VMEM((32,), jnp.int32),
            pltpu.VMEM((32,), jnp.int32),
            pltpu.VMEM((_BPW,), jnp.int32),
            pltpu.VMEM((_BPW,), jnp.int32),
            pltpu.VMEM((_BPW,), jnp.int32),
            pltpu.VMEM((32, 2 * _DIM), jnp.float32),
            pltpu.VMEM((48, 2 * _DIM), jnp.float32),
            pltpu.VMEM((48, 2 * _DIM), jnp.float32),
            pltpu.VMEM((48, 2 * _DIM), jnp.float32),
            pltpu.VMEM((32, 2 * _DIM), jnp.float32),
            pltpu.VMEM((32, 2 * _DIM), jnp.float32),
            pltpu.VMEM((_BPW, 2 * _DIM), jnp.float32),
            pltpu.VMEM((_BPW, 2 * _DIM), jnp.float32),
            pltpu.VMEM((_BPW, 2 * _DIM), jnp.float32),
            pltpu.VMEM((_PROWS, 128), jnp.float32),
            pltpu.SemaphoreType.DMA,
            pltpu.SemaphoreType.DMA,
        ],
    )
    def _sc_loss(ce_hbm, re_hbm, nf1_hbm, nf2_hbm, nf3_hbm, nf4_hbm,
                 dis_hbm, neg_hbm, top_hbm, rel_hbm, out_hbm,
                 i1, i2, i3, i4, i5, i6, i7, ir3, ir4,
                 r1v, r2v, r3v, r4v, r5v, r6v, r7v, rr3, rr4, buf, sem, isem):
        w = lax.axis_index("s") * 2 + lax.axis_index("c")

        ic1 = pltpu.async_copy(nf1_hbm.at[pl.ds(32 * w, 32)], i1, isem)
        ic2 = pltpu.async_copy(nf2_hbm.at[pl.ds(48 * w, 48)], i2, isem)
        ic3 = pltpu.async_copy(nf3_hbm.at[pl.ds(48 * w, 48)], i3, isem)
        icr3 = pltpu.async_copy(rel_hbm.at[0, pl.ds(_BPW * w, _BPW)], ir3, isem)
        ic4 = pltpu.async_copy(nf4_hbm.at[pl.ds(48 * w, 48)], i4, isem)
        icr4 = pltpu.async_copy(rel_hbm.at[1, pl.ds(_BPW * w, _BPW)], ir4, isem)
        ic5 = pltpu.async_copy(dis_hbm.at[pl.ds(32 * w, 32)], i5, isem)
        ic6 = pltpu.async_copy(neg_hbm.at[pl.ds(32 * w, 32)], i6, isem)
        ic7 = pltpu.async_copy(top_hbm.at[pl.ds(_BPW * w, _BPW)], i7, isem)
        ic1.wait()
        g1 = pltpu.async_copy(ce_hbm.at[i1], r1v, sem)
        ic2.wait()
        g2 = pltpu.async_copy(ce_hbm.at[i2], r2v, sem)
        ic3.wait()
        g3 = pltpu.async_copy(ce_hbm.at[i3], r3v, sem)
        icr3.wait()
        gr3 = pltpu.async_copy(re_hbm.at[ir3], rr3, sem)
        ic4.wait()
        g4 = pltpu.async_copy(ce_hbm.at[i4], r4v, sem)
        icr4.wait()
        gr4 = pltpu.async_copy(re_hbm.at[ir4], rr4, sem)
        ic5.wait()
        g5 = pltpu.async_copy(ce_hbm.at[i5], r5v, sem)
        ic6.wait()
        g6 = pltpu.async_copy(ce_hbm.at[i6], r6v, sem)
        ic7.wait()
        g7 = pltpu.async_copy(ce_hbm.at[i7], r7v, sem)

        m = jnp.float32(_MARGIN)
        m1 = jnp.float32(_MARGIN1)
        lanes = lax.iota(jnp.int32, _L)

        def halves(ref, row, g):
            a = ref[row, pl.ds(g * _L, _L)]
            b = jnp.abs(ref[row, pl.ds(_DIM + g * _L, _L)])
            return a, b

        def put(row, acc_or_parts):
            if isinstance(acc_or_parts, list):
                acc = acc_or_parts[0] * acc_or_parts[0]
                for q in acc_or_parts[1:]:
                    acc = acc + q * q
            else:
                acc = acc_or_parts
            buf[row // 8, pl.ds((row % 8) * _L, _L)] = acc

        def head_2op(base, ref, sgn_r, bias):
            def body(i, _):
                t1 = []
                t2 = []
                t3 = []
                for g in range(8):
                    c1, cr = halves(ref, 2 * i, g)
                    d1, dr = halves(ref, 2 * i + 1, g)
                    euc = jnp.abs(c1 - d1)
                    if sgn_r == 0:
                        t = euc + cr - dr + bias
                    elif sgn_r == 1:
                        t = cr + dr + bias - euc
                    else:
                        t = euc - cr - dr + bias
                    t1.append(jnp.maximum(t, 0.0))
                    t2.append(jnp.maximum(m - cr, 0.0))
                    t3.append(jnp.maximum(m - dr, 0.0))
                put(base + i, t1)
                put(base + _BPW + i, t2)
                put(base + 2 * _BPW + i, t3)
                return 0

            lax.fori_loop(0, _BPW, body, 0, unroll=False)

        g1.wait()
        head_2op(0, r1v, 0, m1)

        def nf2_body(i, _):
            t1 = []
            t2 = []
            for g in range(8):
                c1, c2 = halves(r2v, 3 * i, g)
                d1, d2 = halves(r2v, 3 * i + 1, g)
                e1, er = halves(r2v, 3 * i + 2, g)
                start = jnp.maximum(c1 - c2, d1 - d2)
                end = jnp.minimum(c1 + c2, d1 + d2)
                new_r = (end - start) * 0.5
                cen = (start + end) * 0.5
                euc = jnp.abs(cen - e1)
                t1.append(jnp.maximum(euc + new_r - er + m1, 0.0))
                t2.append(jnp.maximum(start - end, 0.0))
            put(9 * _BPW + i, t1)
            put(10 * _BPW + i, t2)
            return 0

        g2.wait()
        lax.fori_loop(0, _BPW, nf2_body, 0, unroll=False)

        def rel_head(base, ref, rref, crow, drow, sgn):
            def body(i, _):
                dtail = rref[i, pl.ds(_DIM - _L + 1, _L)]
                delta = jnp.abs(dtail[_L - 1])
                bias = m1 - delta if sgn > 0 else m1 + delta
                t1 = []
                t2 = []
                t3 = []
                for g in range(8):
                    c1, cr = halves(ref, 3 * i + crow, g)
                    d1, dr = halves(ref, 3 * i + drow, g)
                    r1 = rref[i, pl.ds(g * _L, _L)]
                    euc = jnp.abs(c1 + r1 - d1) if sgn > 0 else jnp.abs(c1 - r1 - d1)
                    if sgn > 0:
                        t = euc + cr - dr + bias
                    else:
                        t = euc - cr - dr + bias
                    t1.append(jnp.maximum(t, 0.0))
                    t2.append(jnp.maximum(m - cr, 0.0))
                    t3.append(jnp.maximum(m - dr, 0.0))
                put(base + i, t1)
                put(base + _BPW + i, t2)
                put(base + 2 * _BPW + i, t3)
                put(base + 3 * _BPW + i,
                    jnp.where(lanes == _L - 1, dtail * dtail, 0.0))
                return 0

            lax.fori_loop(0, _BPW, body, 0, unroll=False)

        g3.wait()
        gr3.wait()
        rel_head(11 * _BPW, r3v, rr3, 0, 2, 1)
        g4.wait()
        gr4.wait()
        rel_head(15 * _BPW, r4v, rr4, 1, 2, -1)

        g5.wait()
        head_2op(3 * _BPW, r5v, 1, m1)
        g6.wait()
        head_2op(6 * _BPW, r6v, -1, -m1)

        def top_body(i, _):
            t1 = []
            t2 = []
            for g in range(8):
                d1, dr = halves(r7v, i, g)
                t1.append(jnp.maximum(_INF - dr * 0.5, 0.0))
                t2.append(jnp.maximum(_INF + d1, 0.0))
            put(19 * _BPW + i, t1)
            put(20 * _BPW + i, t2)
            return 0

        g7.wait()
        lax.fori_loop(0, _BPW, top_body, 0, unroll=False)

        zero = jnp.zeros((_L,), jnp.float32)
        for r in range(_ROWS, _PROWS * 8):
            buf[r // 8, pl.ds((r % 8) * _L, _L)] = zero
        pltpu.sync_copy(buf, out_hbm.at[pl.ds(w * _PROWS, _PROWS)])

    return _sc_loss


def _finalize_body(p_ref, out_ref):
    x = p_ref[...]
    col = lax.broadcasted_iota(jnp.int32, (128, 8), 0) // _L
    grp = lax.broadcasted_iota(jnp.int32, (128, 8), 1)
    sel = (col == grp).astype(jnp.float32)
    x_hi = x.astype(jnp.bfloat16).astype(jnp.float32)
    x_lo = x - x_hi
    dn = (((1,), (0,)), ((), ()))
    sums = (lax.dot_general(x_hi, sel, dn, preferred_element_type=jnp.float32)
            + lax.dot_general(x_lo, sel, dn, preferred_element_type=jnp.float32))
    out_ref[0, 0] = jnp.sum(jnp.sqrt(sums)) * (1.0 / _B)


def _finalize(partials):
    return pl.pallas_call(
        _finalize_body,
        out_shape=jax.ShapeDtypeStruct((1, 1), jnp.float32),
        out_specs=pl.BlockSpec(memory_space=pltpu.SMEM),
    )(partials)


def kernel(class_emb, rel_emb, nf1, nf2, nf3, nf4, disjoint, neg, top):
    re_pad = jnp.pad(rel_emb, ((0, 0), (0, 2 * _DIM - (_DIM + 1))))
    rel_cols = jnp.stack([nf3[:_B, 1], nf4[:_B, 0]])
    partials = _get_sc_kernel()(
        class_emb, re_pad,
        nf1.reshape(-1), nf2.reshape(-1), nf3.reshape(-1), nf4.reshape(-1),
        disjoint.reshape(-1), neg.reshape(-1), top, rel_cols)
    return _finalize(partials)[0, 0]
